# baseline (speedup 1.0000x reference)
import functools

import jax
import jax.numpy as jnp
from jax import lax
from jax.experimental import pallas as pl
from jax.experimental.pallas import tpu as pltpu
from jax.experimental.pallas import tpu_sc as plsc

N = 10000
E = 160000
EP = 163840
D_IN = 256
D_OUT = 256
CW = D_OUT // 2
NC = 2
NH = 4
HROWS = N // (NC * NH)
AROWS = HROWS + 14
NT = 16
EPT = EP // NT
BK = 64
CAP = EPT + NH * 2 * BK
LANES = 16
SRPT = AROWS // NT
ZROWS = 32
WTILES = 10
WRPT = HROWS // WTILES
ZROWS2 = 32


_BM = 1000


def _mm_body(x_ref, w_ref, b_ref, o_ref):
    h = lax.dot_general(
        x_ref[...], w_ref[...],
        (((1,), (1,)), ((), ())),
        preferred_element_type=jnp.float32,
    )
    h = h + b_ref[...]
    lo = lax.bitcast_convert_type(h[:, :CW], jnp.int32)
    hi = lax.bitcast_convert_type(h[:, CW:], jnp.int32)
    lo = lo + 0x7FFF + ((lo >> 16) & 1)
    hi = hi + 0x7FFF + ((hi >> 16) & 1)
    o_ref[...] = ((lo >> 16) & 0xFFFF) | (hi & jnp.int32(-65536))


def _matmul(x, w, b2):
    return pl.pallas_call(
        _mm_body,
        grid=(N // _BM,),
        in_specs=[
            pl.BlockSpec((_BM, D_IN), lambda i: (i, 0)),
            pl.BlockSpec((D_OUT, D_IN), lambda i: (0, 0)),
            pl.BlockSpec((1, D_OUT), lambda i: (0, 0)),
        ],
        out_specs=pl.BlockSpec((_BM, CW), lambda i: (i, 0)),
        out_shape=jax.ShapeDtypeStruct((N, CW), jnp.int32),
    )(x, w, b2)


_mesh = plsc.VectorSubcoreMesh(core_axis_name="c", subcore_axis_name="s")


def _masks(d, qbase):
    ge = [d >= qbase + k * HROWS for k in range(NH + 1)]
    return tuple(ge[k] & ~ge[k + 1] for k in range(NH))


def _popcnt(m):
    return jnp.max(plsc.all_reduce_population_count(m))


@functools.partial(
    pl.kernel,
    out_type=jax.ShapeDtypeStruct((N, D_OUT), jnp.float32),
    mesh=_mesh,
    compiler_params=pltpu.CompilerParams(
        needs_layout_passes=False, use_tc_tiling_on_sc=False),
    scratch_types=[
        pltpu.VMEM((EPT,), jnp.int32),
        pltpu.VMEM((EPT,), jnp.int32),
        pltpu.VMEM((EPT,), jnp.float32),
        pltpu.VMEM((CAP,), jnp.int32),
        pltpu.VMEM((CAP,), jnp.int32),
        pltpu.VMEM((CAP,), jnp.float32),
        pltpu.VMEM((2, BK, CW), jnp.int32),
        pltpu.VMEM((BK, D_OUT), jnp.float32),
        pltpu.VMEM((ZROWS, D_OUT), jnp.float32),
        pltpu.VMEM_SHARED((AROWS, D_OUT), jnp.float32),
        pltpu.SemaphoreType.DMA,
        pltpu.SemaphoreType.DMA,
    ],
)
def _sc_spmm(hpk, src2, dst2, w2, out, rsrc, rdst, rw, csrc, cdst, cwgt,
             msg, msgf, zbuf, acc, gsem0, gsem1):
    cid = lax.axis_index("c")
    sid = lax.axis_index("s")
    gsems = (gsem0, gsem1)
    qbase = cid * (NH * HROWS)

    pltpu.async_copy(src2.at[sid], rsrc, gsem0)
    pltpu.async_copy(dst2.at[sid], rdst, gsem0)
    pltpu.async_copy(w2.at[sid], rw, gsem0)
    for _ in range(3):
        pltpu.make_async_copy(src2.at[0], rsrc, gsem0).wait()

    dummy = jnp.full((LANES,), HROWS, jnp.int32)
    zero_i = jnp.zeros((LANES,), jnp.int32)
    zero_f = jnp.zeros((LANES,), jnp.float32)

    def _fill(i, _):
        sl = pl.ds(i * LANES, LANES)
        csrc[sl] = zero_i
        cdst[sl] = dummy
        cwgt[sl] = zero_f
        return 0
    lax.fori_loop(0, CAP // LANES, _fill, 0)

    def _cgrp(g, cnts):
        d = rdst[pl.ds(g * LANES, LANES)]
        ms = _masks(d, qbase)
        return tuple(cnts[h] + _popcnt(ms[h]) for h in range(NH))

    counts = lax.fori_loop(0, EPT // LANES, _cgrp,
                           tuple(jnp.int32(0) for _ in range(NH)))

    starts = [jnp.int32(0)]
    for h in range(1, NH):
        starts.append((starts[h - 1] + counts[h - 1] + 127) & (-128))

    def _sgrp(g, offs):
        sl = pl.ds(g * LANES, LANES)
        s = rsrc[sl]
        d = rdst[sl]
        wt = rw[sl]
        ms = _masks(d, qbase)
        offs_new = []
        for h in range(NH):
            off = offs[h]
            win = pl.ds(off, LANES)
            plsc.store_compressed(csrc.at[win], s, mask=ms[h])
            plsc.store_compressed(cdst.at[win], d - (qbase + h * HROWS),
                                  mask=ms[h])
            plsc.store_compressed(cwgt.at[win], wt, mask=ms[h])
            offs_new.append(off + _popcnt(ms[h]))
        return tuple(offs_new)

    lax.fori_loop(0, EPT // LANES, _sgrp, tuple(starts))

    def _zrow(i, _):
        def _zg(g, _):
            zbuf[i, pl.ds(g * LANES, LANES)] = zero_f
            return 0
        return lax.fori_loop(0, D_OUT // LANES, _zg, 0)
    lax.fori_loop(0, ZROWS, _zrow, 0)

    for h in range(NH):
        base = starts[h]
        npair = (counts[h] + 2 * BK - 1) >> 7

        for i in range(SRPT // ZROWS):
            pltpu.sync_copy(zbuf, acc.at[pl.ds(sid * SRPT + i * ZROWS,
                                               ZROWS)])
        if SRPT % ZROWS:
            pltpu.sync_copy(
                zbuf.at[pl.ds(0, SRPT % ZROWS)],
                acc.at[pl.ds(sid * SRPT + (SRPT // ZROWS) * ZROWS,
                             SRPT % ZROWS)])

        plsc.subcore_barrier()

        @pl.when(npair > 0)
        def _():
            pltpu.async_copy(hpk.at[csrc.at[pl.ds(pl.multiple_of(base, BK),
                                                  BK)]], msg.at[0], gsem0)

        def _pair(jj, _):
            for b in range(2):
                j = jj * 2 + b
                pltpu.make_async_copy(
                    hpk.at[csrc.at[pl.ds(pl.multiple_of(base + j * BK, BK),
                                         BK)]],
                    msg.at[b], gsems[b]).wait()

                @pl.when(j + 1 < npair * 2)
                def _():
                    pltpu.async_copy(
                        hpk.at[csrc.at[pl.ds(
                            pl.multiple_of(base + (j + 1) * BK, BK), BK)]],
                        msg.at[1 - b], gsems[1 - b])

                def _scale(e, _):
                    wbc = plsc.load_gather(
                        cwgt, [jnp.full((LANES,), base + j * BK + e,
                                        jnp.int32)])
                    for g in range(CW // LANES):
                        sl = pl.ds(g * LANES, LANES)
                        sh = pl.ds(CW + g * LANES, LANES)
                        v = msg[b, e, sl]
                        lo = plsc.bitcast(lax.shift_left(v, 16), jnp.float32)
                        hi = plsc.bitcast(v & jnp.int32(-65536), jnp.float32)
                        msgf[e, sl] = lo * wbc
                        msgf[e, sh] = hi * wbc
                    return 0
                lax.fori_loop(0, BK, _scale, 0)

                pltpu.sync_copy(
                    msgf,
                    acc.at[cdst.at[pl.ds(pl.multiple_of(base + j * BK, BK),
                                         BK)]], add=True)
            return 0

        lax.fori_loop(0, npair, _pair, 0)

        plsc.subcore_barrier()

        @pl.when(sid < WTILES)
        def _():
            pltpu.sync_copy(
                acc.at[pl.ds(sid * WRPT, WRPT)],
                out.at[pl.ds(qbase + h * HROWS + sid * WRPT, WRPT)])

        plsc.subcore_barrier()


def kernel(x, edge_index, edge_weight, W, b):
    hpk = _matmul(x, W, b.reshape(1, D_OUT))
    pad = jnp.zeros((EP - E,), jnp.int32)
    src2 = jnp.concatenate([edge_index[1], pad]).reshape(NT, EPT)
    dst2 = jnp.concatenate([edge_index[0], pad]).reshape(NT, EPT)
    w2 = jnp.concatenate(
        [edge_weight, jnp.zeros((EP - E,), jnp.float32)]).reshape(NT, EPT)
    return _sc_spmm(hpk, src2, dst2, w2)


# speedup vs baseline: 1.5596x; 1.5596x over previous
import functools

import jax
import jax.numpy as jnp
from jax import lax
from jax.experimental import pallas as pl
from jax.experimental.pallas import tpu as pltpu
from jax.experimental.pallas import tpu_sc as plsc

N = 10000
E = 160000
EP = 163840
D_IN = 256
D_OUT = 256
NC = 2
CH = D_OUT // NC
CW = CH // 2
NH = 2
HR = N // NH
AROWS = HR + 8
NT = 16
EPT = EP // NT
BK = 64
CAP = EPT + NH * 2 * BK
LANES = 16
SRPT = AROWS // NT
ZROWS = 32
WTILES = 10
WRPT = HR // WTILES


_BM = 1000


def _mm_body(x_ref, w_ref, b_ref, o_ref):
    h = lax.dot_general(
        x_ref[...], w_ref[...],
        (((1,), (1,)), ((), ())),
        preferred_element_type=jnp.float32,
    )
    h = h + b_ref[0]
    lo = lax.bitcast_convert_type(h[:, :CW], jnp.int32)
    hi = lax.bitcast_convert_type(h[:, CW:], jnp.int32)
    lo = lo + 0x7FFF + ((lo >> 16) & 1)
    hi = hi + 0x7FFF + ((hi >> 16) & 1)
    o_ref[...] = (((lo >> 16) & 0xFFFF) | (hi & jnp.int32(-65536)))[None]


def _matmul(x, w, b3):
    return pl.pallas_call(
        _mm_body,
        grid=(NC, N // _BM),
        in_specs=[
            pl.BlockSpec((_BM, D_IN), lambda c, i: (i, 0)),
            pl.BlockSpec((CH, D_IN), lambda c, i: (c, 0)),
            pl.BlockSpec((1, 1, CH), lambda c, i: (c, 0, 0)),
        ],
        out_specs=pl.BlockSpec((1, _BM, CW), lambda c, i: (c, i, 0)),
        out_shape=jax.ShapeDtypeStruct((NC, N, CW), jnp.int32),
    )(x, w, b3)


_mesh = plsc.VectorSubcoreMesh(core_axis_name="c", subcore_axis_name="s")


def _popcnt(m):
    return jnp.max(plsc.all_reduce_population_count(m))


@functools.partial(
    pl.kernel,
    out_type=jax.ShapeDtypeStruct((NC, N, CH), jnp.float32),
    mesh=_mesh,
    compiler_params=pltpu.CompilerParams(
        needs_layout_passes=False, use_tc_tiling_on_sc=False),
    scratch_types=[
        pltpu.VMEM((EPT,), jnp.int32),
        pltpu.VMEM((EPT,), jnp.int32),
        pltpu.VMEM((EPT,), jnp.float32),
        pltpu.VMEM((CAP,), jnp.int32),
        pltpu.VMEM((CAP,), jnp.int32),
        pltpu.VMEM((CAP,), jnp.float32),
        pltpu.VMEM((2, BK, CW), jnp.int32),
        pltpu.VMEM((BK, CH), jnp.float32),
        pltpu.VMEM((ZROWS, CH), jnp.float32),
        pltpu.VMEM_SHARED((AROWS, CH), jnp.float32),
        pltpu.SemaphoreType.DMA,
        pltpu.SemaphoreType.DMA,
    ],
)
def _sc_spmm(hpk, src2, dst2, w2, out, rsrc, rdst, rw, csrc, cdst, cwgt,
             msg, msgf, zbuf, acc, gsem0, gsem1):
    cid = lax.axis_index("c")
    sid = lax.axis_index("s")
    gsems = (gsem0, gsem1)
    hc = hpk.at[cid]

    pltpu.async_copy(src2.at[sid], rsrc, gsem0)
    pltpu.async_copy(dst2.at[sid], rdst, gsem0)
    pltpu.async_copy(w2.at[sid], rw, gsem0)
    for _ in range(3):
        pltpu.make_async_copy(src2.at[0], rsrc, gsem0).wait()

    dummy = jnp.full((LANES,), HR, jnp.int32)
    zero_i = jnp.zeros((LANES,), jnp.int32)
    zero_f = jnp.zeros((LANES,), jnp.float32)

    def _fill(i, _):
        sl = pl.ds(i * LANES, LANES)
        csrc[sl] = zero_i
        cdst[sl] = dummy
        cwgt[sl] = zero_f
        return 0
    lax.fori_loop(0, CAP // LANES, _fill, 0)

    def _cgrp(g, cnts):
        d = rdst[pl.ds(g * LANES, LANES)]
        m1 = d >= HR
        return (cnts[0] + _popcnt(~m1), cnts[1] + _popcnt(m1))

    counts = lax.fori_loop(0, EPT // LANES, _cgrp,
                           (jnp.int32(0), jnp.int32(0)))
    starts = (jnp.int32(0), (counts[0] + 127) & (-128))

    def _sgrp(g, offs):
        sl = pl.ds(g * LANES, LANES)
        s = rsrc[sl]
        d = rdst[sl]
        wt = rw[sl]
        m1 = d >= HR
        ms = (~m1, m1)
        offs_new = []
        for h in range(NH):
            off = offs[h]
            win = pl.ds(off, LANES)
            plsc.store_compressed(csrc.at[win], s, mask=ms[h])
            plsc.store_compressed(cdst.at[win], d - h * HR, mask=ms[h])
            plsc.store_compressed(cwgt.at[win], wt, mask=ms[h])
            offs_new.append(off + _popcnt(ms[h]))
        return tuple(offs_new)

    lax.fori_loop(0, EPT // LANES, _sgrp, starts)

    def _zrow(i, _):
        def _zg(g, _):
            zbuf[i, pl.ds(g * LANES, LANES)] = zero_f
            return 0
        return lax.fori_loop(0, CH // LANES, _zg, 0)
    lax.fori_loop(0, ZROWS, _zrow, 0)

    for h in range(NH):
        base = starts[h]
        npair = (counts[h] + 2 * BK - 1) >> 7

        for i in range(SRPT // ZROWS):
            pltpu.sync_copy(zbuf, acc.at[pl.ds(sid * SRPT + i * ZROWS,
                                               ZROWS)])
        if SRPT % ZROWS:
            pltpu.sync_copy(
                zbuf.at[pl.ds(0, SRPT % ZROWS)],
                acc.at[pl.ds(sid * SRPT + (SRPT // ZROWS) * ZROWS,
                             SRPT % ZROWS)])

        plsc.subcore_barrier()

        @pl.when(npair > 0)
        def _():
            pltpu.async_copy(hc.at[csrc.at[pl.ds(pl.multiple_of(base, BK),
                                                 BK)]], msg.at[0], gsem0)

        def _pair(jj, _):
            for b in range(2):
                j = jj * 2 + b
                pltpu.make_async_copy(
                    hc.at[csrc.at[pl.ds(pl.multiple_of(base + j * BK, BK),
                                        BK)]],
                    msg.at[b], gsems[b]).wait()

                @pl.when(j + 1 < npair * 2)
                def _():
                    pltpu.async_copy(
                        hc.at[csrc.at[pl.ds(
                            pl.multiple_of(base + (j + 1) * BK, BK), BK)]],
                        msg.at[1 - b], gsems[1 - b])

                def _scale(e, _):
                    wbc = plsc.load_gather(
                        cwgt, [jnp.full((LANES,), base + j * BK + e,
                                        jnp.int32)])
                    for g in range(CW // LANES):
                        sl = pl.ds(g * LANES, LANES)
                        sh = pl.ds(CW + g * LANES, LANES)
                        v = msg[b, e, sl]
                        lo = plsc.bitcast(lax.shift_left(v, 16), jnp.float32)
                        hi = plsc.bitcast(v & jnp.int32(-65536), jnp.float32)
                        msgf[e, sl] = lo * wbc
                        msgf[e, sh] = hi * wbc
                    return 0
                lax.fori_loop(0, BK, _scale, 0)

                pltpu.sync_copy(
                    msgf,
                    acc.at[cdst.at[pl.ds(pl.multiple_of(base + j * BK, BK),
                                         BK)]], add=True)
            return 0

        lax.fori_loop(0, npair, _pair, 0)

        plsc.subcore_barrier()

        @pl.when(sid < WTILES)
        def _():
            pltpu.sync_copy(acc.at[pl.ds(sid * WRPT, WRPT)],
                            out.at[cid, pl.ds(h * HR + sid * WRPT, WRPT)])

        plsc.subcore_barrier()


def kernel(x, edge_index, edge_weight, W, b):
    hpk = _matmul(x, W, b.reshape(NC, 1, CH))
    pad = jnp.zeros((EP - E,), jnp.int32)
    src2 = jnp.concatenate([edge_index[1], pad]).reshape(NT, EPT)
    dst2 = jnp.concatenate([edge_index[0], pad]).reshape(NT, EPT)
    w2 = jnp.concatenate(
        [edge_weight, jnp.zeros((EP - E,), jnp.float32)]).reshape(NT, EPT)
    out = _sc_spmm(hpk, src2, dst2, w2)
    return out.transpose(1, 0, 2).reshape(N, D_OUT)


# speedup vs baseline: 1.6801x; 1.0773x over previous
import functools

import jax
import jax.numpy as jnp
from jax import lax
from jax.experimental import pallas as pl
from jax.experimental.pallas import tpu as pltpu
from jax.experimental.pallas import tpu_sc as plsc

N = 10000
E = 160000
D_IN = 256
D_OUT = 256
CH = 64
NCHUNK = D_OUT // CH
NC = 2
NPASS = NCHUNK // NC
NT = 16
EPT = E // NT
BK = 40
NB = EPT // BK
RPT = N // NT
ZR = 125
LANES = 16


_BM = 1000


def _mm_body(x_ref, w_ref, b_ref, o_ref):
    h = lax.dot_general(
        x_ref[...], w_ref[...],
        (((1,), (1,)), ((), ())),
        preferred_element_type=jnp.float32,
    )
    o_ref[...] = (h + b_ref[0])[None]


def _matmul(x, w, b2):
    return pl.pallas_call(
        _mm_body,
        grid=(NCHUNK, N // _BM),
        in_specs=[
            pl.BlockSpec((_BM, D_IN), lambda c, i: (i, 0)),
            pl.BlockSpec((CH, D_IN), lambda c, i: (c, 0)),
            pl.BlockSpec((1, 1, CH), lambda c, i: (c, 0, 0)),
        ],
        out_specs=pl.BlockSpec((1, _BM, CH), lambda c, i: (c, i, 0)),
        out_shape=jax.ShapeDtypeStruct((NCHUNK, N, CH), jnp.float32),
    )(x, w, b2)


_mesh = plsc.VectorSubcoreMesh(core_axis_name="c", subcore_axis_name="s")


@functools.partial(
    pl.kernel,
    out_type=jax.ShapeDtypeStruct((N, NCHUNK, CH), jnp.float32),
    mesh=_mesh,
    compiler_params=pltpu.CompilerParams(
        needs_layout_passes=False, use_tc_tiling_on_sc=False),
    scratch_types=[
        pltpu.VMEM((NB, BK), jnp.int32),
        pltpu.VMEM((NB, BK), jnp.int32),
        pltpu.VMEM((EPT,), jnp.float32),
        pltpu.VMEM((2, BK, CH), jnp.float32),
        pltpu.VMEM((ZR, CH), jnp.float32),
        pltpu.VMEM_SHARED((N, CH), jnp.float32),
        pltpu.SemaphoreType.DMA,
        pltpu.SemaphoreType.DMA,
    ],
)
def _sc_spmm(hblk, src3, dst3, w2, out, srcv, dstv, wv, msg, zbuf, acc,
             gsem0, gsem1):
    cid = lax.axis_index("c")
    sid = lax.axis_index("s")
    gsems = (gsem0, gsem1)

    pltpu.sync_copy(src3.at[sid], srcv)
    pltpu.sync_copy(dst3.at[sid], dstv)
    pltpu.sync_copy(w2.at[sid], wv)

    def _zrow(i, _):
        def _zg(g, _):
            zbuf[i, pl.ds(g * LANES, LANES)] = jnp.zeros((LANES,), jnp.float32)
            return 0
        return lax.fori_loop(0, CH // LANES, _zg, 0)
    lax.fori_loop(0, ZR, _zrow, 0)

    for p in range(NPASS):
        chunk = cid * NPASS + p
        hc = hblk.at[chunk]

        def _zcp(i, _):
            pltpu.sync_copy(zbuf, acc.at[pl.ds(sid * RPT + i * ZR, ZR)])
            return 0
        lax.fori_loop(0, RPT // ZR, _zcp, 0)

        plsc.subcore_barrier()

        pltpu.async_copy(hc.at[srcv.at[0]], msg.at[0], gsem0)

        def _pair(jj, _):
            for b in range(2):
                j = jj * 2 + b
                pltpu.make_async_copy(hc.at[srcv.at[j]], msg.at[b],
                                      gsems[b]).wait()

                @pl.when(j + 1 < NB)
                def _():
                    pltpu.async_copy(hc.at[srcv.at[j + 1]], msg.at[1 - b],
                                     gsems[1 - b])

                def _scale(e, _):
                    wbc = plsc.load_gather(
                        wv, [jnp.full((LANES,), j * BK + e, jnp.int32)])
                    for g in range(CH // LANES):
                        sl = pl.ds(g * LANES, LANES)
                        msg[b, e, sl] = msg[b, e, sl] * wbc
                    return 0
                lax.fori_loop(0, BK, _scale, 0)

                pltpu.sync_copy(msg.at[b], acc.at[dstv.at[j]], add=True)
            return 0

        lax.fori_loop(0, NB // 2, _pair, 0)

        plsc.subcore_barrier()

        pltpu.sync_copy(acc.at[pl.ds(sid * RPT, RPT)],
                        out.at[pl.ds(sid * RPT, RPT), chunk])


def kernel(x, edge_index, edge_weight, W, b):
    hblk = _matmul(x, W, b.reshape(NCHUNK, 1, CH))
    src3 = edge_index[1].reshape(NT, NB, BK)
    dst3 = edge_index[0].reshape(NT, NB, BK)
    w2 = edge_weight.reshape(NT, EPT)
    out = _sc_spmm(hblk, src3, dst3, w2)
    return out.reshape(N, D_OUT)
